# Initial kernel scaffold; baseline (speedup 1.0000x reference)
import jax
import jax.numpy as jnp
from jax.experimental import pallas as pl


def kernel(h, edge_index, W0, al0, ar0, g0, b0, W1, al1, ar1, g1, b1, W2, al2, ar2, g2, b2, W3, al3, ar3, g3, b3, P1, pb1, P2, pb2):
    raise NotImplementedError("write your pallas kernel here")


# speedup vs baseline: 1.0825x; 1.0825x over previous
import functools

import jax
import jax.numpy as jnp
from jax.experimental import pallas as pl
from jax.experimental.pallas import tpu as pltpu

N = 10000
E = 320000
D = 128
LCFG = [(128, 8, 16), (128, 8, 16), (128, 8, 16), (128, 1, 128)]


def _vmem(n=None):
    return pl.BlockSpec(memory_space=pltpu.ANY) if n is None else pl.BlockSpec(memory_space=pltpu.VMEM)


def _dense0_body(h_ref, w_ref, al_ref, ar_ref, z_ref, el_ref, er_ref):
    z = jnp.dot(h_ref[...], w_ref[...], preferred_element_type=jnp.float32)
    z_ref[...] = z
    el_ref[...] = jnp.dot(z, al_ref[...], preferred_element_type=jnp.float32)
    er_ref[...] = jnp.dot(z, ar_ref[...], preferred_element_type=jnp.float32)


@jax.jit
def _dense0(h, w, al16, ar16):
    return pl.pallas_call(
        _dense0_body,
        out_shape=[
            jax.ShapeDtypeStruct((N, D), jnp.float32),
            jax.ShapeDtypeStruct((N, 16), jnp.float32),
            jax.ShapeDtypeStruct((N, 16), jnp.float32),
        ],
    )(h, w, al16, ar16)


def _epi_dense_body(ms_ref, es_ref, sel_ref, g_ref, b_ref, hin_ref,
                    w_ref, al_ref, ar_ref, h_ref, z_ref, el_ref, er_ref):
    den = jnp.dot(es_ref[...], sel_ref[...], preferred_element_type=jnp.float32)
    rst = ms_ref[...] / (den + 1e-9)
    mean = jnp.mean(rst, axis=0, keepdims=True)
    var = jnp.mean(rst * rst, axis=0, keepdims=True) - mean * mean
    rst = (rst - mean) * jax.lax.rsqrt(var + 1e-5) * g_ref[...] + b_ref[...]
    rst = jnp.where(rst > 0, rst, jnp.exp(jnp.minimum(rst, 0.0)) - 1.0)
    hout = hin_ref[...] + rst
    h_ref[...] = hout
    z = jnp.dot(hout, w_ref[...], preferred_element_type=jnp.float32)
    z_ref[...] = z
    el_ref[...] = jnp.dot(z, al_ref[...], preferred_element_type=jnp.float32)
    er_ref[...] = jnp.dot(z, ar_ref[...], preferred_element_type=jnp.float32)


@jax.jit
def _epi_dense(msum, esum, sel, g, b, hin, w, al16, ar16):
    return pl.pallas_call(
        _epi_dense_body,
        out_shape=[
            jax.ShapeDtypeStruct((N, D), jnp.float32),
            jax.ShapeDtypeStruct((N, D), jnp.float32),
            jax.ShapeDtypeStruct((N, 16), jnp.float32),
            jax.ShapeDtypeStruct((N, 16), jnp.float32),
        ],
    )(msum, esum, sel, g.reshape(1, D), b.reshape(1, D), hin, w, al16, ar16)


def _epi_final_body(ms_ref, es_ref, sel_ref, g_ref, b_ref, hin_ref,
                    p1a_ref, p1b_ref, pb1_ref, a_ref, bout_ref):
    den = jnp.dot(es_ref[...], sel_ref[...], preferred_element_type=jnp.float32)
    rst = ms_ref[...] / (den + 1e-9)
    mean = jnp.mean(rst, axis=0, keepdims=True)
    var = jnp.mean(rst * rst, axis=0, keepdims=True) - mean * mean
    rst = (rst - mean) * jax.lax.rsqrt(var + 1e-5) * g_ref[...] + b_ref[...]
    rst = jnp.where(rst > 0, rst, jnp.exp(jnp.minimum(rst, 0.0)) - 1.0)
    hout = hin_ref[...] + rst
    a_ref[...] = jnp.dot(hout, p1a_ref[...], preferred_element_type=jnp.float32)
    bout_ref[...] = (jnp.dot(hout, p1b_ref[...], preferred_element_type=jnp.float32)
                     + pb1_ref[...])


@jax.jit
def _epi_final(msum, esum, sel, g, b, hin, p1a, p1b, pb1):
    return pl.pallas_call(
        _epi_final_body,
        out_shape=[
            jax.ShapeDtypeStruct((N, D), jnp.float32),
            jax.ShapeDtypeStruct((N, D), jnp.float32),
        ],
    )(msum, esum, sel, g.reshape(1, D), b.reshape(1, D), hin, p1a, p1b,
      pb1.reshape(1, D))


def _edge_pass_jax(z, el16, er16, src, dst, hds, od):
    el = el16[:, :hds]
    er = er16[:, :hds]
    e = el[src] + er[dst]
    e = jnp.where(e > 0, e, 0.2 * e)
    ex = jnp.exp(e)
    esum = jax.ops.segment_sum(ex, dst, num_segments=N)
    zh = z.reshape(N, hds, od)
    msg = ex[:, :, None] * zh[src]
    msum = jax.ops.segment_sum(msg, dst, num_segments=N).reshape(N, hds * od)
    esum16 = jnp.zeros((N, 16), jnp.float32).at[:, :hds].set(esum)
    return msum, esum16


def _final_edge_jax(a, b, src, dst, p2, pb2):
    x = jnp.maximum(a[src] + b[dst], 0.0)
    return jnp.dot(x, p2) + pb2


def _pack_attn(al, ar, hds, od):
    al16 = jnp.zeros((D, 16), jnp.float32)
    ar16 = jnp.zeros((D, 16), jnp.float32)
    for h in range(hds):
        al16 = al16.at[h * od:(h + 1) * od, h].set(al[h])
        ar16 = ar16.at[h * od:(h + 1) * od, h].set(ar[h])
    sel = jnp.zeros((16, D), jnp.float32)
    for h in range(hds):
        sel = sel.at[h, h * od:(h + 1) * od].set(1.0)
    return al16, ar16, sel


def kernel(h, edge_index, W0, al0, ar0, g0, b0, W1, al1, ar1, g1, b1,
           W2, al2, ar2, g2, b2, W3, al3, ar3, g3, b3, P1, pb1, P2, pb2):
    src = edge_index[0]
    dst = edge_index[1]
    params = [(W0, al0, ar0, g0, b0), (W1, al1, ar1, g1, b1),
              (W2, al2, ar2, g2, b2), (W3, al3, ar3, g3, b3)]

    packed = [_pack_attn(al, ar, hds, od)
              for (ind, hds, od), (_, al, ar, _, _) in zip(LCFG, params)]

    hin = h
    al16, ar16, _ = packed[0]
    z, el16, er16 = _dense0(hin, W0, al16, ar16)
    for i in range(4):
        ind, hds, od = LCFG[i]
        _, _, _, g, b = params[i]
        msum, esum16 = _edge_pass_jax(z, el16, er16, src, dst, hds, od)
        sel = packed[i][2]
        if i < 3:
            al16n, ar16n, _ = packed[i + 1]
            hin, z, el16, er16 = _epi_dense(msum, esum16, sel, g, b, hin,
                                            params[i + 1][0], al16n, ar16n)
        else:
            a, bb = _epi_final(msum, esum16, sel, g, b, hin,
                               P1[:D], P1[D:], pb1)
    out = _final_edge_jax(a, bb, src, dst, P2, pb2)
    return out


# speedup vs baseline: 30.8852x; 28.5307x over previous
import functools

import jax
import jax.numpy as jnp
from jax import lax
from jax.experimental import pallas as pl
from jax.experimental.pallas import tpu as pltpu
from jax.experimental.pallas import tpu_sc as plsc

N = 10000
E = 320000
D = 128
LCFG = [(128, 8, 16), (128, 8, 16), (128, 8, 16), (128, 1, 128)]

_MESH = plsc.VectorSubcoreMesh(core_axis_name="c", subcore_axis_name="s")
_SC_PARAMS = pltpu.CompilerParams(use_tc_tiling_on_sc=False,
                                  needs_layout_passes=False)
NTILES = 32
EPT = E // NTILES
CH = 80
NCH = EPT // CH
R0 = 624
REM = N - 16 * R0


def _vmem(n=None):
    return pl.BlockSpec(memory_space=pltpu.ANY) if n is None else pl.BlockSpec(memory_space=pltpu.VMEM)


def _dense0_body(h_ref, w_ref, al_ref, ar_ref, z_ref, el_ref, er_ref):
    z = jnp.dot(h_ref[...], w_ref[...], preferred_element_type=jnp.float32)
    z_ref[...] = z
    el_ref[...] = jnp.dot(z, al_ref[...], preferred_element_type=jnp.float32)
    er_ref[...] = jnp.dot(z, ar_ref[...], preferred_element_type=jnp.float32)


@jax.jit
def _dense0(h, w, al16, ar16):
    return pl.pallas_call(
        _dense0_body,
        out_shape=[
            jax.ShapeDtypeStruct((N, D), jnp.float32),
            jax.ShapeDtypeStruct((N, 16), jnp.float32),
            jax.ShapeDtypeStruct((N, 16), jnp.float32),
        ],
    )(h, w, al16, ar16)


def _epi_dense_body(ms_ref, es_ref, sel_ref, g_ref, b_ref, hin_ref,
                    w_ref, al_ref, ar_ref, h_ref, z_ref, el_ref, er_ref):
    es = es_ref[0] + es_ref[1]
    den = jnp.dot(es, sel_ref[...], preferred_element_type=jnp.float32)
    rst = (ms_ref[0] + ms_ref[1]) / (den + 1e-9)
    mean = jnp.mean(rst, axis=0, keepdims=True)
    var = jnp.mean(rst * rst, axis=0, keepdims=True) - mean * mean
    rst = (rst - mean) * jax.lax.rsqrt(var + 1e-5) * g_ref[...] + b_ref[...]
    rst = jnp.where(rst > 0, rst, jnp.exp(jnp.minimum(rst, 0.0)) - 1.0)
    hout = hin_ref[...] + rst
    h_ref[...] = hout
    z = jnp.dot(hout, w_ref[...], preferred_element_type=jnp.float32)
    z_ref[...] = z
    el_ref[...] = jnp.dot(z, al_ref[...], preferred_element_type=jnp.float32)
    er_ref[...] = jnp.dot(z, ar_ref[...], preferred_element_type=jnp.float32)


@jax.jit
def _epi_dense(msum, esum, sel, g, b, hin, w, al16, ar16):
    return pl.pallas_call(
        _epi_dense_body,
        out_shape=[
            jax.ShapeDtypeStruct((N, D), jnp.float32),
            jax.ShapeDtypeStruct((N, D), jnp.float32),
            jax.ShapeDtypeStruct((N, 16), jnp.float32),
            jax.ShapeDtypeStruct((N, 16), jnp.float32),
        ],
    )(msum, esum, sel, g.reshape(1, D), b.reshape(1, D), hin, w, al16, ar16)


def _epi_final_body(ms_ref, es_ref, sel_ref, g_ref, b_ref, hin_ref,
                    p1a_ref, p1b_ref, pb1_ref, a_ref, bout_ref):
    es = es_ref[0] + es_ref[1]
    den = jnp.dot(es, sel_ref[...], preferred_element_type=jnp.float32)
    rst = (ms_ref[0] + ms_ref[1]) / (den + 1e-9)
    mean = jnp.mean(rst, axis=0, keepdims=True)
    var = jnp.mean(rst * rst, axis=0, keepdims=True) - mean * mean
    rst = (rst - mean) * jax.lax.rsqrt(var + 1e-5) * g_ref[...] + b_ref[...]
    rst = jnp.where(rst > 0, rst, jnp.exp(jnp.minimum(rst, 0.0)) - 1.0)
    hout = hin_ref[...] + rst
    a_ref[...] = jnp.dot(hout, p1a_ref[...], preferred_element_type=jnp.float32)
    bout_ref[...] = (jnp.dot(hout, p1b_ref[...], preferred_element_type=jnp.float32)
                     + pb1_ref[...])


@jax.jit
def _epi_final(msum, esum, sel, g, b, hin, p1a, p1b, pb1):
    return pl.pallas_call(
        _epi_final_body,
        out_shape=[
            jax.ShapeDtypeStruct((N, D), jnp.float32),
            jax.ShapeDtypeStruct((N, D), jnp.float32),
        ],
    )(msum, esum, sel, g.reshape(1, D), b.reshape(1, D), hin, p1a, p1b,
      pb1.reshape(1, D))


def _zero16(ref, rows):
    def body(i, _):
        for t in range(ref.shape[1] // 16):
            ref[i, pl.ds(t * 16, 16)] = jnp.zeros((16,), jnp.float32)
        return 0
    lax.fori_loop(0, rows, body, 0, unroll=4)


def _edge_body(hds, od, z_hbm, el_hbm, er_hbm, src_hbm, dst_hbm,
               outm_hbm, oute_hbm,
               srcv, dstv, elg, erg, zg, msg, exb,
               accm, acce, sem0, sem1, sem2):
    c = lax.axis_index("c")
    s = lax.axis_index("s")
    g = c * 16 + s
    _zero16(msg, CH)
    _zero16(exb, CH)
    base = s * R0
    nfull = R0 // CH
    rem = R0 - nfull * CH
    for k in range(nfull):
        pltpu.sync_copy(msg, accm.at[pl.ds(base + k * CH, CH)])
        pltpu.sync_copy(exb, acce.at[pl.ds(base + k * CH, CH)])
    if rem:
        pltpu.sync_copy(msg.at[pl.ds(0, rem)],
                        accm.at[pl.ds(base + nfull * CH, rem)])
        pltpu.sync_copy(exb.at[pl.ds(0, rem)],
                        acce.at[pl.ds(base + nfull * CH, rem)])

    @pl.when(s == 15)
    def _zero_tail():
        pltpu.sync_copy(msg.at[pl.ds(0, REM)], accm.at[pl.ds(16 * R0, REM)])
        pltpu.sync_copy(exb.at[pl.ds(0, REM)], acce.at[pl.ds(16 * R0, REM)])
    plsc.subcore_barrier()

    def chunk(j, _):
        pltpu.sync_copy(src_hbm.at[g].at[j], srcv)
        pltpu.sync_copy(dst_hbm.at[g].at[j], dstv)
        cp1 = pltpu.async_copy(el_hbm.at[srcv], elg, sem0)
        cp2 = pltpu.async_copy(er_hbm.at[dstv], erg, sem1)
        cp3 = pltpu.async_copy(z_hbm.at[srcv], zg, sem2)
        cp1.wait()
        cp2.wait()
        cp3.wait()

        def edge(i, _):
            a = elg[i, :] + erg[i, :]
            a = jnp.maximum(a, 0.2 * a)
            ex = jnp.exp(a)
            exb[i, :] = ex
            for h in range(hds):
                sv = ex[h]
                for t in range(od // 16):
                    off = h * od + t * 16
                    msg[i, pl.ds(off, 16)] = sv * zg[i, pl.ds(off, 16)]
            return 0
        lax.fori_loop(0, CH, edge, 0)
        pltpu.sync_copy(msg, accm.at[dstv], add=True)
        pltpu.sync_copy(exb, acce.at[dstv], add=True)
        return 0
    lax.fori_loop(0, NCH, chunk, 0)
    plsc.subcore_barrier()
    pltpu.sync_copy(accm.at[pl.ds(base, R0)],
                    outm_hbm.at[c].at[pl.ds(base, R0)])
    pltpu.sync_copy(acce.at[pl.ds(base, R0)],
                    oute_hbm.at[c].at[pl.ds(base, R0)])

    @pl.when(s == 15)
    def _out_tail():
        pltpu.sync_copy(accm.at[pl.ds(16 * R0, REM)],
                        outm_hbm.at[c].at[pl.ds(16 * R0, REM)])
        pltpu.sync_copy(acce.at[pl.ds(16 * R0, REM)],
                        oute_hbm.at[c].at[pl.ds(16 * R0, REM)])


def _make_edge_pass(hds, od):
    return pl.kernel(
        functools.partial(_edge_body, hds, od),
        out_type=[
            jax.ShapeDtypeStruct((2, N, D), jnp.float32),
            jax.ShapeDtypeStruct((2, N, 16), jnp.float32),
        ],
        mesh=_MESH,
        compiler_params=_SC_PARAMS,
        scratch_types=[
            pltpu.VMEM((CH,), jnp.int32),
            pltpu.VMEM((CH,), jnp.int32),
            pltpu.VMEM((CH, 16), jnp.float32),
            pltpu.VMEM((CH, 16), jnp.float32),
            pltpu.VMEM((CH, D), jnp.float32),
            pltpu.VMEM((CH, D), jnp.float32),
            pltpu.VMEM((CH, 16), jnp.float32),
            pltpu.VMEM_SHARED((N, D), jnp.float32),
            pltpu.VMEM_SHARED((N, 16), jnp.float32),
            pltpu.SemaphoreType.DMA,
            pltpu.SemaphoreType.DMA,
            pltpu.SemaphoreType.DMA,
        ],
    )


_edge_pass_h8 = jax.jit(_make_edge_pass(8, 16))
_edge_pass_h1 = jax.jit(_make_edge_pass(1, 128))


def _final_body(a_hbm, b_hbm, src_hbm, dst_hbm, p2_hbm, pb2_hbm, out_hbm,
                srcv, dstv, ag, bg, ob, p2v, pb2v, sem0, sem1):
    c = lax.axis_index("c")
    s = lax.axis_index("s")
    g = c * 16 + s
    pltpu.sync_copy(src_hbm.at[g], srcv)
    pltpu.sync_copy(dst_hbm.at[g], dstv)
    pltpu.sync_copy(p2_hbm, p2v)
    pltpu.sync_copy(pb2_hbm, pb2v)
    pb2vec = pb2v[:]
    lane = lax.iota(jnp.int32, 16)

    def chunk(j, _):
        cp1 = pltpu.async_copy(a_hbm.at[srcv.at[j]], ag, sem0)
        cp2 = pltpu.async_copy(b_hbm.at[dstv.at[j]], bg, sem1)
        cp1.wait()
        cp2.wait()

        def edge(i, _):
            acc0 = jnp.zeros((16,), jnp.float32)
            acc1 = jnp.zeros((16,), jnp.float32)
            for t in range(D // 16):
                r = jnp.maximum(ag[i, pl.ds(t * 16, 16)]
                                + bg[i, pl.ds(t * 16, 16)], 0.0)
                acc0 = acc0 + r * p2v[0, pl.ds(t * 16, 16)]
                acc1 = acc1 + r * p2v[1, pl.ds(t * 16, 16)]
            s0 = jnp.sum(acc0) + pb2vec[0]
            s1 = jnp.sum(acc1) + pb2vec[1]
            v = jnp.where(lane < 1, s0, s1)
            rowi = jnp.full((16,), i, jnp.int32)
            coli = jnp.minimum(lane, 1)
            plsc.store_scatter(ob, [rowi, coli], v, mask=lane < 2)
            return 0
        lax.fori_loop(0, CH, edge, 0)
        pltpu.sync_copy(ob, out_hbm.at[pl.ds(g * EPT + j * CH, CH)])
        return 0
    lax.fori_loop(0, NCH, chunk, 0)


_final_edge = jax.jit(pl.kernel(
    _final_body,
    out_type=jax.ShapeDtypeStruct((E, 2), jnp.float32),
    mesh=_MESH,
    compiler_params=_SC_PARAMS,
    scratch_types=[
        pltpu.VMEM((NCH, CH), jnp.int32),
        pltpu.VMEM((NCH, CH), jnp.int32),
        pltpu.VMEM((CH, D), jnp.float32),
        pltpu.VMEM((CH, D), jnp.float32),
        pltpu.VMEM((CH, 2), jnp.float32),
        pltpu.VMEM((2, D), jnp.float32),
        pltpu.VMEM((16,), jnp.float32),
        pltpu.SemaphoreType.DMA,
        pltpu.SemaphoreType.DMA,
    ],
))


def _pack_attn(al, ar, hds, od):
    al16 = jnp.zeros((D, 16), jnp.float32)
    ar16 = jnp.zeros((D, 16), jnp.float32)
    for h in range(hds):
        al16 = al16.at[h * od:(h + 1) * od, h].set(al[h])
        ar16 = ar16.at[h * od:(h + 1) * od, h].set(ar[h])
    sel = jnp.zeros((16, D), jnp.float32)
    for h in range(hds):
        sel = sel.at[h, h * od:(h + 1) * od].set(1.0)
    return al16, ar16, sel


def kernel(h, edge_index, W0, al0, ar0, g0, b0, W1, al1, ar1, g1, b1,
           W2, al2, ar2, g2, b2, W3, al3, ar3, g3, b3, P1, pb1, P2, pb2):
    src3 = edge_index[0].reshape(NTILES, NCH, CH)
    dst3 = edge_index[1].reshape(NTILES, NCH, CH)
    params = [(W0, al0, ar0, g0, b0), (W1, al1, ar1, g1, b1),
              (W2, al2, ar2, g2, b2), (W3, al3, ar3, g3, b3)]

    packed = [_pack_attn(al, ar, hds, od)
              for (ind, hds, od), (_, al, ar, _, _) in zip(LCFG, params)]

    hin = h
    al16, ar16, _ = packed[0]
    z, el16, er16 = _dense0(hin, W0, al16, ar16)
    for i in range(4):
        ind, hds, od = LCFG[i]
        _, _, _, g, b = params[i]
        edge_pass = _edge_pass_h8 if hds == 8 else _edge_pass_h1
        msum, esum16 = edge_pass(z, el16, er16, src3, dst3)
        sel = packed[i][2]
        if i < 3:
            al16n, ar16n, _ = packed[i + 1]
            hin, z, el16, er16 = _epi_dense(msum, esum16, sel, g, b, hin,
                                            params[i + 1][0], al16n, ar16n)
        else:
            a, bb = _epi_final(msum, esum16, sel, g, b, hin,
                               P1[:D], P1[D:], pb1)
    p2t = P2.T
    pb2p = jnp.zeros((16,), jnp.float32).at[:2].set(pb2)
    out = _final_edge(a, bb, src3, dst3, p2t, pb2p)
    return out


# speedup vs baseline: 42.2891x; 1.3692x over previous
import functools

import jax
import jax.numpy as jnp
from jax import lax
from jax.experimental import pallas as pl
from jax.experimental.pallas import tpu as pltpu
from jax.experimental.pallas import tpu_sc as plsc

N = 10000
E = 320000
D = 128
LCFG = [(128, 8, 16), (128, 8, 16), (128, 8, 16), (128, 1, 128)]

_MESH = plsc.VectorSubcoreMesh(core_axis_name="c", subcore_axis_name="s")
_SC_PARAMS = pltpu.CompilerParams(use_tc_tiling_on_sc=False,
                                  needs_layout_passes=False)
NTILES = 32
EPT = E // NTILES
CH = 80
NCH = EPT // CH
R0 = 624
REM = N - 16 * R0


def _vmem(n=None):
    return pl.BlockSpec(memory_space=pltpu.ANY) if n is None else pl.BlockSpec(memory_space=pltpu.VMEM)


def _dense0_body(h_ref, w_ref, al_ref, ar_ref, z_ref, el_ref, er_ref):
    z = jnp.dot(h_ref[...], w_ref[...], preferred_element_type=jnp.float32)
    z_ref[...] = z
    el_ref[...] = jnp.dot(z, al_ref[...], preferred_element_type=jnp.float32)
    er_ref[...] = jnp.dot(z, ar_ref[...], preferred_element_type=jnp.float32)


@jax.jit
def _dense0(h, w, al16, ar16):
    return pl.pallas_call(
        _dense0_body,
        out_shape=[
            jax.ShapeDtypeStruct((N, D), jnp.float32),
            jax.ShapeDtypeStruct((N, 16), jnp.float32),
            jax.ShapeDtypeStruct((N, 16), jnp.float32),
        ],
    )(h, w, al16, ar16)


def _epi_dense_body(ms_ref, es_ref, sel_ref, g_ref, b_ref, hin_ref,
                    w_ref, al_ref, ar_ref, h_ref, z_ref, el_ref, er_ref):
    es = es_ref[0] + es_ref[1]
    den = jnp.dot(es, sel_ref[...], preferred_element_type=jnp.float32)
    rst = (ms_ref[0] + ms_ref[1]) / (den + 1e-9)
    mean = jnp.mean(rst, axis=0, keepdims=True)
    var = jnp.mean(rst * rst, axis=0, keepdims=True) - mean * mean
    rst = (rst - mean) * jax.lax.rsqrt(var + 1e-5) * g_ref[...] + b_ref[...]
    rst = jnp.where(rst > 0, rst, jnp.exp(jnp.minimum(rst, 0.0)) - 1.0)
    hout = hin_ref[...] + rst
    h_ref[...] = hout
    z = jnp.dot(hout, w_ref[...], preferred_element_type=jnp.float32)
    z_ref[...] = z
    el_ref[...] = jnp.dot(z, al_ref[...], preferred_element_type=jnp.float32)
    er_ref[...] = jnp.dot(z, ar_ref[...], preferred_element_type=jnp.float32)


@jax.jit
def _epi_dense(msum, esum, sel, g, b, hin, w, al16, ar16):
    return pl.pallas_call(
        _epi_dense_body,
        out_shape=[
            jax.ShapeDtypeStruct((N, D), jnp.float32),
            jax.ShapeDtypeStruct((N, D), jnp.float32),
            jax.ShapeDtypeStruct((N, 16), jnp.float32),
            jax.ShapeDtypeStruct((N, 16), jnp.float32),
        ],
    )(msum, esum, sel, g.reshape(1, D), b.reshape(1, D), hin, w, al16, ar16)


def _epi_final_body(ms_ref, es_ref, sel_ref, g_ref, b_ref, hin_ref,
                    p1a_ref, p1b_ref, pb1_ref, a_ref, bout_ref):
    es = es_ref[0] + es_ref[1]
    den = jnp.dot(es, sel_ref[...], preferred_element_type=jnp.float32)
    rst = (ms_ref[0] + ms_ref[1]) / (den + 1e-9)
    mean = jnp.mean(rst, axis=0, keepdims=True)
    var = jnp.mean(rst * rst, axis=0, keepdims=True) - mean * mean
    rst = (rst - mean) * jax.lax.rsqrt(var + 1e-5) * g_ref[...] + b_ref[...]
    rst = jnp.where(rst > 0, rst, jnp.exp(jnp.minimum(rst, 0.0)) - 1.0)
    hout = hin_ref[...] + rst
    a_ref[...] = jnp.dot(hout, p1a_ref[...], preferred_element_type=jnp.float32)
    bout_ref[...] = (jnp.dot(hout, p1b_ref[...], preferred_element_type=jnp.float32)
                     + pb1_ref[...])


@jax.jit
def _epi_final(msum, esum, sel, g, b, hin, p1a, p1b, pb1):
    return pl.pallas_call(
        _epi_final_body,
        out_shape=[
            jax.ShapeDtypeStruct((N, D), jnp.float32),
            jax.ShapeDtypeStruct((N, D), jnp.float32),
        ],
    )(msum, esum, sel, g.reshape(1, D), b.reshape(1, D), hin, p1a, p1b,
      pb1.reshape(1, D))


def _zero16(ref, rows):
    def body(i, _):
        for t in range(ref.shape[1] // 16):
            ref[i, pl.ds(t * 16, 16)] = jnp.zeros((16,), jnp.float32)
        return 0
    lax.fori_loop(0, rows, body, 0, unroll=4)


def _edge_body(hds, od, z_hbm, el_hbm, er_hbm, src_hbm, dst_hbm,
               outm_hbm, oute_hbm,
               srcv0, srcv1, dstv0, dstv1, dsts, elg0, elg1, erg0, erg1,
               zg0, zg1, msg, exb,
               accm, acce, semi0, semi1, semg0, semg1):
    srcv = [srcv0, srcv1]
    dstv = [dstv0, dstv1]
    elg = [elg0, elg1]
    erg = [erg0, erg1]
    zg = [zg0, zg1]
    semi = [semi0, semi1]
    semg = [semg0, semg1]
    c = lax.axis_index("c")
    s = lax.axis_index("s")
    g = c * 16 + s
    _zero16(msg, CH)
    _zero16(exb, CH)
    base = s * R0
    nfull = R0 // CH
    rem = R0 - nfull * CH
    for k in range(nfull):
        pltpu.sync_copy(msg, accm.at[pl.ds(base + k * CH, CH)])
        pltpu.sync_copy(exb, acce.at[pl.ds(base + k * CH, CH)])
    if rem:
        pltpu.sync_copy(msg.at[pl.ds(0, rem)],
                        accm.at[pl.ds(base + nfull * CH, rem)])
        pltpu.sync_copy(exb.at[pl.ds(0, rem)],
                        acce.at[pl.ds(base + nfull * CH, rem)])

    @pl.when(s == 15)
    def _zero_tail():
        pltpu.sync_copy(msg.at[pl.ds(0, REM)], accm.at[pl.ds(16 * R0, REM)])
        pltpu.sync_copy(exb.at[pl.ds(0, REM)], acce.at[pl.ds(16 * R0, REM)])
    plsc.subcore_barrier()

    def issue_idx(j, p):
        pltpu.async_copy(src_hbm.at[g].at[j], srcv[p], semi[p])
        pltpu.async_copy(dst_hbm.at[g].at[j], dstv[p], semi[p])

    def wait_idx(p):
        pltpu.make_async_copy(src_hbm.at[g].at[0], srcv[p], semi[p]).wait()
        pltpu.make_async_copy(dst_hbm.at[g].at[0], dstv[p], semi[p]).wait()

    def issue_gathers(p):
        pltpu.async_copy(el_hbm.at[srcv[p]], elg[p], semg[p])
        pltpu.async_copy(er_hbm.at[dstv[p]], erg[p], semg[p])
        pltpu.async_copy(z_hbm.at[srcv[p]], zg[p], semg[p])

    def wait_gathers(p):
        pltpu.make_async_copy(el_hbm.at[srcv[p]], elg[p], semg[p]).wait()
        pltpu.make_async_copy(er_hbm.at[dstv[p]], erg[p], semg[p]).wait()
        pltpu.make_async_copy(z_hbm.at[srcv[p]], zg[p], semg[p]).wait()

    def compute_scatter(p):
        def edge(i, _):
            a = elg[p][i, :] + erg[p][i, :]
            a = jnp.maximum(a, 0.2 * a)
            ex = jnp.exp(a)
            exb[i, :] = ex
            for h in range(hds):
                sv = ex[h]
                for t in range(od // 16):
                    off = h * od + t * 16
                    msg[i, pl.ds(off, 16)] = sv * zg[p][i, pl.ds(off, 16)]
            return 0
        lax.fori_loop(0, CH, edge, 0, unroll=2)
        pltpu.sync_copy(msg, accm.at[dsts], add=True)
        pltpu.sync_copy(exb, acce.at[dsts], add=True)

    pltpu.sync_copy(src_hbm.at[g].at[0], srcv[0])
    pltpu.sync_copy(dst_hbm.at[g].at[0], dstv[0])
    issue_gathers(0)
    issue_idx(1, 1)

    def body2(t, _):
        for half in range(2):
            p = half
            np = 1 - half
            j = 2 * t + half

            @pl.when(j < NCH)
            def _run():
                @pl.when(j + 1 < NCH)
                def _pf():
                    wait_idx(np)
                    issue_gathers(np)
                wait_gathers(p)
                for k in range(CH // 16):
                    dsts[pl.ds(k * 16, 16)] = dstv[p][pl.ds(k * 16, 16)]

                @pl.when(j + 2 < NCH)
                def _pi():
                    issue_idx(j + 2, p)
                compute_scatter(p)
        return 0
    lax.fori_loop(0, (NCH + 1) // 2, body2, 0)
    plsc.subcore_barrier()
    pltpu.sync_copy(accm.at[pl.ds(base, R0)],
                    outm_hbm.at[c].at[pl.ds(base, R0)])
    pltpu.sync_copy(acce.at[pl.ds(base, R0)],
                    oute_hbm.at[c].at[pl.ds(base, R0)])

    @pl.when(s == 15)
    def _out_tail():
        pltpu.sync_copy(accm.at[pl.ds(16 * R0, REM)],
                        outm_hbm.at[c].at[pl.ds(16 * R0, REM)])
        pltpu.sync_copy(acce.at[pl.ds(16 * R0, REM)],
                        oute_hbm.at[c].at[pl.ds(16 * R0, REM)])


def _make_edge_pass(hds, od):
    return pl.kernel(
        functools.partial(_edge_body, hds, od),
        out_type=[
            jax.ShapeDtypeStruct((2, N, D), jnp.float32),
            jax.ShapeDtypeStruct((2, N, 16), jnp.float32),
        ],
        mesh=_MESH,
        compiler_params=_SC_PARAMS,
        scratch_types=[
            pltpu.VMEM((CH,), jnp.int32),
            pltpu.VMEM((CH,), jnp.int32),
            pltpu.VMEM((CH,), jnp.int32),
            pltpu.VMEM((CH,), jnp.int32),
            pltpu.VMEM((CH,), jnp.int32),
            pltpu.VMEM((CH, 16), jnp.float32),
            pltpu.VMEM((CH, 16), jnp.float32),
            pltpu.VMEM((CH, 16), jnp.float32),
            pltpu.VMEM((CH, 16), jnp.float32),
            pltpu.VMEM((CH, D), jnp.float32),
            pltpu.VMEM((CH, D), jnp.float32),
            pltpu.VMEM((CH, D), jnp.float32),
            pltpu.VMEM((CH, 16), jnp.float32),
            pltpu.VMEM_SHARED((N, D), jnp.float32),
            pltpu.VMEM_SHARED((N, 16), jnp.float32),
            pltpu.SemaphoreType.DMA,
            pltpu.SemaphoreType.DMA,
            pltpu.SemaphoreType.DMA,
            pltpu.SemaphoreType.DMA,
        ],
    )


_edge_pass_h8 = jax.jit(_make_edge_pass(8, 16))
_edge_pass_h1 = jax.jit(_make_edge_pass(1, 128))


def _final_body(a_hbm, b_hbm, src_hbm, dst_hbm, p2_hbm, pb2_hbm, out_hbm,
                srcv, dstv, ag0, ag1, bg0, bg1, ob0, ob1, p2v, pb2v,
                semg0, semg1, semo0, semo1):
    ag = [ag0, ag1]
    bg = [bg0, bg1]
    ob = [ob0, ob1]
    semg = [semg0, semg1]
    semo = [semo0, semo1]
    c = lax.axis_index("c")
    s = lax.axis_index("s")
    g = c * 16 + s
    pltpu.sync_copy(src_hbm.at[g], srcv)
    pltpu.sync_copy(dst_hbm.at[g], dstv)
    pltpu.sync_copy(p2_hbm, p2v)
    pltpu.sync_copy(pb2_hbm, pb2v)
    pb2vec = pb2v[:]
    lane = lax.iota(jnp.int32, 16)

    def issue_g(j, p):
        pltpu.async_copy(a_hbm.at[srcv.at[j]], ag[p], semg[p])
        pltpu.async_copy(b_hbm.at[dstv.at[j]], bg[p], semg[p])

    def wait_g(p):
        pltpu.make_async_copy(a_hbm.at[srcv.at[0]], ag[p], semg[p]).wait()
        pltpu.make_async_copy(b_hbm.at[dstv.at[0]], bg[p], semg[p]).wait()

    def wait_o(p):
        pltpu.make_async_copy(ob[p], out_hbm.at[pl.ds(g * EPT, CH)],
                              semo[p]).wait()

    def compute(j, p):
        def edge(i, _):
            acc0 = jnp.zeros((16,), jnp.float32)
            acc1 = jnp.zeros((16,), jnp.float32)
            for t in range(D // 16):
                r = jnp.maximum(ag[p][i, pl.ds(t * 16, 16)]
                                + bg[p][i, pl.ds(t * 16, 16)], 0.0)
                acc0 = acc0 + r * p2v[0, pl.ds(t * 16, 16)]
                acc1 = acc1 + r * p2v[1, pl.ds(t * 16, 16)]
            s0 = jnp.sum(acc0) + pb2vec[0]
            s1 = jnp.sum(acc1) + pb2vec[1]
            v = jnp.where(lane < 1, s0, s1)
            rowi = jnp.full((16,), i, jnp.int32)
            coli = jnp.minimum(lane, 1)
            plsc.store_scatter(ob[p], [rowi, coli], v, mask=lane < 2)
            return 0
        lax.fori_loop(0, CH, edge, 0, unroll=2)
        pltpu.async_copy(ob[p], out_hbm.at[pl.ds(g * EPT + j * CH, CH)],
                         semo[p])

    issue_g(0, 0)

    def body2(t, _):
        for half in range(2):
            p = half
            np = 1 - half
            j = 2 * t + half

            @pl.when(j < NCH)
            def _run():
                @pl.when(j + 1 < NCH)
                def _pf():
                    issue_g(j + 1, np)
                wait_g(p)

                @pl.when(j >= 2)
                def _wo():
                    wait_o(p)
                compute(j, p)
        return 0
    lax.fori_loop(0, (NCH + 1) // 2, body2, 0)
    wait_o(1)
    wait_o(0)


_final_edge = jax.jit(pl.kernel(
    _final_body,
    out_type=jax.ShapeDtypeStruct((E, 2), jnp.float32),
    mesh=_MESH,
    compiler_params=_SC_PARAMS,
    scratch_types=[
        pltpu.VMEM((NCH, CH), jnp.int32),
        pltpu.VMEM((NCH, CH), jnp.int32),
        pltpu.VMEM((CH, D), jnp.float32),
        pltpu.VMEM((CH, D), jnp.float32),
        pltpu.VMEM((CH, D), jnp.float32),
        pltpu.VMEM((CH, D), jnp.float32),
        pltpu.VMEM((CH, 2), jnp.float32),
        pltpu.VMEM((CH, 2), jnp.float32),
        pltpu.VMEM((2, D), jnp.float32),
        pltpu.VMEM((16,), jnp.float32),
        pltpu.SemaphoreType.DMA,
        pltpu.SemaphoreType.DMA,
        pltpu.SemaphoreType.DMA,
        pltpu.SemaphoreType.DMA,
    ],
))


def _pack_attn(al, ar, hds, od):
    al16 = jnp.zeros((D, 16), jnp.float32)
    ar16 = jnp.zeros((D, 16), jnp.float32)
    for h in range(hds):
        al16 = al16.at[h * od:(h + 1) * od, h].set(al[h])
        ar16 = ar16.at[h * od:(h + 1) * od, h].set(ar[h])
    sel = jnp.zeros((16, D), jnp.float32)
    for h in range(hds):
        sel = sel.at[h, h * od:(h + 1) * od].set(1.0)
    return al16, ar16, sel


def kernel(h, edge_index, W0, al0, ar0, g0, b0, W1, al1, ar1, g1, b1,
           W2, al2, ar2, g2, b2, W3, al3, ar3, g3, b3, P1, pb1, P2, pb2):
    src3 = edge_index[0].reshape(NTILES, NCH, CH)
    dst3 = edge_index[1].reshape(NTILES, NCH, CH)
    params = [(W0, al0, ar0, g0, b0), (W1, al1, ar1, g1, b1),
              (W2, al2, ar2, g2, b2), (W3, al3, ar3, g3, b3)]

    packed = [_pack_attn(al, ar, hds, od)
              for (ind, hds, od), (_, al, ar, _, _) in zip(LCFG, params)]

    hin = h
    al16, ar16, _ = packed[0]
    z, el16, er16 = _dense0(hin, W0, al16, ar16)
    for i in range(4):
        ind, hds, od = LCFG[i]
        _, _, _, g, b = params[i]
        edge_pass = _edge_pass_h8 if hds == 8 else _edge_pass_h1
        msum, esum16 = edge_pass(z, el16, er16, src3, dst3)
        sel = packed[i][2]
        if i < 3:
            al16n, ar16n, _ = packed[i + 1]
            hin, z, el16, er16 = _epi_dense(msum, esum16, sel, g, b, hin,
                                            params[i + 1][0], al16n, ar16n)
        else:
            a, bb = _epi_final(msum, esum16, sel, g, b, hin,
                               P1[:D], P1[D:], pb1)
    p2t = P2.T
    pb2p = jnp.zeros((16,), jnp.float32).at[:2].set(pb2)
    out = _final_edge(a, bb, src3, dst3, p2t, pb2p)
    return out


# speedup vs baseline: 42.5880x; 1.0071x over previous
import functools

import jax
import jax.numpy as jnp
from jax import lax
from jax.experimental import pallas as pl
from jax.experimental.pallas import tpu as pltpu
from jax.experimental.pallas import tpu_sc as plsc

N = 10000
E = 320000
D = 128
LCFG = [(128, 8, 16), (128, 8, 16), (128, 8, 16), (128, 1, 128)]

_MESH = plsc.VectorSubcoreMesh(core_axis_name="c", subcore_axis_name="s")
_SC_PARAMS = pltpu.CompilerParams(use_tc_tiling_on_sc=False,
                                  needs_layout_passes=False)
NTILES = 32
EPT = E // NTILES
CH = 80
NCH = EPT // CH
R0 = 624
REM = N - 16 * R0


def _vmem(n=None):
    return pl.BlockSpec(memory_space=pltpu.ANY) if n is None else pl.BlockSpec(memory_space=pltpu.VMEM)


def _dense0_body(h_ref, w_ref, al_ref, ar_ref, z_ref, el_ref, er_ref):
    z = jnp.dot(h_ref[...], w_ref[...], preferred_element_type=jnp.float32)
    z_ref[...] = z
    el_ref[...] = jnp.dot(z, al_ref[...], preferred_element_type=jnp.float32)
    er_ref[...] = jnp.dot(z, ar_ref[...], preferred_element_type=jnp.float32)


@jax.jit
def _dense0(h, w, al16, ar16):
    return pl.pallas_call(
        _dense0_body,
        out_shape=[
            jax.ShapeDtypeStruct((N, D), jnp.float32),
            jax.ShapeDtypeStruct((N, 16), jnp.float32),
            jax.ShapeDtypeStruct((N, 16), jnp.float32),
        ],
    )(h, w, al16, ar16)


def _epi_dense_body(ms_ref, es_ref, sel_ref, g_ref, b_ref, hin_ref,
                    w_ref, al_ref, ar_ref, h_ref, z_ref, el_ref, er_ref):
    es = es_ref[0] + es_ref[1]
    den = jnp.dot(es, sel_ref[...], preferred_element_type=jnp.float32)
    rst = (ms_ref[0] + ms_ref[1]) / (den + 1e-9)
    mean = jnp.mean(rst, axis=0, keepdims=True)
    var = jnp.mean(rst * rst, axis=0, keepdims=True) - mean * mean
    rst = (rst - mean) * jax.lax.rsqrt(var + 1e-5) * g_ref[...] + b_ref[...]
    rst = jnp.where(rst > 0, rst, jnp.exp(jnp.minimum(rst, 0.0)) - 1.0)
    hout = hin_ref[...] + rst
    h_ref[...] = hout
    z = jnp.dot(hout, w_ref[...], preferred_element_type=jnp.float32)
    z_ref[...] = z
    el_ref[...] = jnp.dot(z, al_ref[...], preferred_element_type=jnp.float32)
    er_ref[...] = jnp.dot(z, ar_ref[...], preferred_element_type=jnp.float32)


@jax.jit
def _epi_dense(msum, esum, sel, g, b, hin, w, al16, ar16):
    return pl.pallas_call(
        _epi_dense_body,
        out_shape=[
            jax.ShapeDtypeStruct((N, D), jnp.float32),
            jax.ShapeDtypeStruct((N, D), jnp.float32),
            jax.ShapeDtypeStruct((N, 16), jnp.float32),
            jax.ShapeDtypeStruct((N, 16), jnp.float32),
        ],
    )(msum, esum, sel, g.reshape(1, D), b.reshape(1, D), hin, w, al16, ar16)


def _epi_final_body(ms_ref, es_ref, sel_ref, g_ref, b_ref, hin_ref,
                    p1a_ref, p1b_ref, pb1_ref, a_ref, bout_ref):
    es = es_ref[0] + es_ref[1]
    den = jnp.dot(es, sel_ref[...], preferred_element_type=jnp.float32)
    rst = (ms_ref[0] + ms_ref[1]) / (den + 1e-9)
    mean = jnp.mean(rst, axis=0, keepdims=True)
    var = jnp.mean(rst * rst, axis=0, keepdims=True) - mean * mean
    rst = (rst - mean) * jax.lax.rsqrt(var + 1e-5) * g_ref[...] + b_ref[...]
    rst = jnp.where(rst > 0, rst, jnp.exp(jnp.minimum(rst, 0.0)) - 1.0)
    hout = hin_ref[...] + rst
    a_ref[...] = jnp.dot(hout, p1a_ref[...], preferred_element_type=jnp.float32)
    bout_ref[...] = (jnp.dot(hout, p1b_ref[...], preferred_element_type=jnp.float32)
                     + pb1_ref[...])


@jax.jit
def _epi_final(msum, esum, sel, g, b, hin, p1a, p1b, pb1):
    return pl.pallas_call(
        _epi_final_body,
        out_shape=[
            jax.ShapeDtypeStruct((N, D), jnp.float32),
            jax.ShapeDtypeStruct((N, D), jnp.float32),
        ],
    )(msum, esum, sel, g.reshape(1, D), b.reshape(1, D), hin, p1a, p1b,
      pb1.reshape(1, D))


def _zero16(ref, rows):
    def body(i, _):
        for t in range(ref.shape[1] // 16):
            ref[i, pl.ds(t * 16, 16)] = jnp.zeros((16,), jnp.float32)
        return 0
    lax.fori_loop(0, rows, body, 0, unroll=4)


def _edge_body(hds, od, z_hbm, el_hbm, er_hbm, src_hbm, dst_hbm,
               outm_hbm, oute_hbm,
               srcv0, srcv1, dstv0, dstv1, dsts, elg0, elg1, erg0, erg1,
               zg0, zg1, msg, exb,
               accm, acce, semi0, semi1, semg0, semg1):
    srcv = [srcv0, srcv1]
    dstv = [dstv0, dstv1]
    elg = [elg0, elg1]
    erg = [erg0, erg1]
    zg = [zg0, zg1]
    semi = [semi0, semi1]
    semg = [semg0, semg1]
    c = lax.axis_index("c")
    s = lax.axis_index("s")
    g = c * 16 + s
    _zero16(msg, CH)
    _zero16(exb, CH)
    base = s * R0
    nfull = R0 // CH
    rem = R0 - nfull * CH
    for k in range(nfull):
        pltpu.sync_copy(msg, accm.at[pl.ds(base + k * CH, CH)])
        pltpu.sync_copy(exb, acce.at[pl.ds(base + k * CH, CH)])
    if rem:
        pltpu.sync_copy(msg.at[pl.ds(0, rem)],
                        accm.at[pl.ds(base + nfull * CH, rem)])
        pltpu.sync_copy(exb.at[pl.ds(0, rem)],
                        acce.at[pl.ds(base + nfull * CH, rem)])

    @pl.when(s == 15)
    def _zero_tail():
        pltpu.sync_copy(msg.at[pl.ds(0, REM)], accm.at[pl.ds(16 * R0, REM)])
        pltpu.sync_copy(exb.at[pl.ds(0, REM)], acce.at[pl.ds(16 * R0, REM)])
    plsc.subcore_barrier()

    def issue_idx(j, p):
        pltpu.async_copy(src_hbm.at[g].at[j], srcv[p], semi[p])
        pltpu.async_copy(dst_hbm.at[g].at[j], dstv[p], semi[p])

    def wait_idx(p):
        pltpu.make_async_copy(src_hbm.at[g].at[0], srcv[p], semi[p]).wait()
        pltpu.make_async_copy(dst_hbm.at[g].at[0], dstv[p], semi[p]).wait()

    def issue_gathers(p):
        pltpu.async_copy(el_hbm.at[srcv[p]], elg[p], semg[p])
        pltpu.async_copy(er_hbm.at[dstv[p]], erg[p], semg[p])
        pltpu.async_copy(z_hbm.at[srcv[p]], zg[p], semg[p])

    def wait_gathers(p):
        pltpu.make_async_copy(el_hbm.at[srcv[p]], elg[p], semg[p]).wait()
        pltpu.make_async_copy(er_hbm.at[dstv[p]], erg[p], semg[p]).wait()
        pltpu.make_async_copy(z_hbm.at[srcv[p]], zg[p], semg[p]).wait()

    gdn = lax.GatherDimensionNumbers(offset_dims=(), collapsed_slice_dims=(0,),
                                     start_index_map=(0,))
    idxh = [jnp.full((16, 1), h, jnp.int32) for h in range(hds)]

    def compute_scatter(p):
        def edge(i, _):
            a = elg[p][i, :] + erg[p][i, :]
            a = jnp.maximum(a, 0.2 * a)
            ex = jnp.exp(a)
            exb[i, :] = ex
            for h in range(hds):
                bv = lax.gather(ex, idxh[h], gdn, slice_sizes=(1,),
                                mode=lax.GatherScatterMode.PROMISE_IN_BOUNDS)
                for t in range(od // 16):
                    off = h * od + t * 16
                    msg[i, pl.ds(off, 16)] = bv * zg[p][i, pl.ds(off, 16)]
            return 0
        lax.fori_loop(0, CH, edge, 0, unroll=4)
        pltpu.sync_copy(msg, accm.at[dsts], add=True)
        pltpu.sync_copy(exb, acce.at[dsts], add=True)

    pltpu.sync_copy(src_hbm.at[g].at[0], srcv[0])
    pltpu.sync_copy(dst_hbm.at[g].at[0], dstv[0])
    issue_gathers(0)
    issue_idx(1, 1)

    def body2(t, _):
        for half in range(2):
            p = half
            np = 1 - half
            j = 2 * t + half

            @pl.when(j < NCH)
            def _run():
                @pl.when(j + 1 < NCH)
                def _pf():
                    wait_idx(np)
                    issue_gathers(np)
                wait_gathers(p)
                for k in range(CH // 16):
                    dsts[pl.ds(k * 16, 16)] = dstv[p][pl.ds(k * 16, 16)]

                @pl.when(j + 2 < NCH)
                def _pi():
                    issue_idx(j + 2, p)
                compute_scatter(p)
        return 0
    lax.fori_loop(0, (NCH + 1) // 2, body2, 0)
    plsc.subcore_barrier()
    pltpu.sync_copy(accm.at[pl.ds(base, R0)],
                    outm_hbm.at[c].at[pl.ds(base, R0)])
    pltpu.sync_copy(acce.at[pl.ds(base, R0)],
                    oute_hbm.at[c].at[pl.ds(base, R0)])

    @pl.when(s == 15)
    def _out_tail():
        pltpu.sync_copy(accm.at[pl.ds(16 * R0, REM)],
                        outm_hbm.at[c].at[pl.ds(16 * R0, REM)])
        pltpu.sync_copy(acce.at[pl.ds(16 * R0, REM)],
                        oute_hbm.at[c].at[pl.ds(16 * R0, REM)])


def _make_edge_pass(hds, od):
    return pl.kernel(
        functools.partial(_edge_body, hds, od),
        out_type=[
            jax.ShapeDtypeStruct((2, N, D), jnp.float32),
            jax.ShapeDtypeStruct((2, N, 16), jnp.float32),
        ],
        mesh=_MESH,
        compiler_params=_SC_PARAMS,
        scratch_types=[
            pltpu.VMEM((CH,), jnp.int32),
            pltpu.VMEM((CH,), jnp.int32),
            pltpu.VMEM((CH,), jnp.int32),
            pltpu.VMEM((CH,), jnp.int32),
            pltpu.VMEM((CH,), jnp.int32),
            pltpu.VMEM((CH, 16), jnp.float32),
            pltpu.VMEM((CH, 16), jnp.float32),
            pltpu.VMEM((CH, 16), jnp.float32),
            pltpu.VMEM((CH, 16), jnp.float32),
            pltpu.VMEM((CH, D), jnp.float32),
            pltpu.VMEM((CH, D), jnp.float32),
            pltpu.VMEM((CH, D), jnp.float32),
            pltpu.VMEM((CH, 16), jnp.float32),
            pltpu.VMEM_SHARED((N, D), jnp.float32),
            pltpu.VMEM_SHARED((N, 16), jnp.float32),
            pltpu.SemaphoreType.DMA,
            pltpu.SemaphoreType.DMA,
            pltpu.SemaphoreType.DMA,
            pltpu.SemaphoreType.DMA,
        ],
    )


_edge_pass_h8 = jax.jit(_make_edge_pass(8, 16))
_edge_pass_h1 = jax.jit(_make_edge_pass(1, 128))


def _final_body(a_hbm, b_hbm, src_hbm, dst_hbm, p2_hbm, pb2_hbm, out_hbm,
                srcv, dstv, ag0, ag1, bg0, bg1, ob0, ob1, p2v, pb2v,
                semg0, semg1, semo0, semo1):
    ag = [ag0, ag1]
    bg = [bg0, bg1]
    ob = [ob0, ob1]
    semg = [semg0, semg1]
    semo = [semo0, semo1]
    c = lax.axis_index("c")
    s = lax.axis_index("s")
    g = c * 16 + s
    pltpu.sync_copy(src_hbm.at[g], srcv)
    pltpu.sync_copy(dst_hbm.at[g], dstv)
    pltpu.sync_copy(p2_hbm, p2v)
    pltpu.sync_copy(pb2_hbm, pb2v)
    pb2vec = pb2v[:]
    lane = lax.iota(jnp.int32, 16)

    def issue_g(j, p):
        pltpu.async_copy(a_hbm.at[srcv.at[j]], ag[p], semg[p])
        pltpu.async_copy(b_hbm.at[dstv.at[j]], bg[p], semg[p])

    def wait_g(p):
        pltpu.make_async_copy(a_hbm.at[srcv.at[0]], ag[p], semg[p]).wait()
        pltpu.make_async_copy(b_hbm.at[dstv.at[0]], bg[p], semg[p]).wait()

    def wait_o(p):
        pltpu.make_async_copy(ob[p], out_hbm.at[pl.ds(g * EPT, CH)],
                              semo[p]).wait()

    def compute(j, p):
        def edge(i, _):
            acc0 = jnp.zeros((16,), jnp.float32)
            acc1 = jnp.zeros((16,), jnp.float32)
            for t in range(D // 16):
                r = jnp.maximum(ag[p][i, pl.ds(t * 16, 16)]
                                + bg[p][i, pl.ds(t * 16, 16)], 0.0)
                acc0 = acc0 + r * p2v[0, pl.ds(t * 16, 16)]
                acc1 = acc1 + r * p2v[1, pl.ds(t * 16, 16)]
            s0 = jnp.sum(acc0) + pb2vec[0]
            s1 = jnp.sum(acc1) + pb2vec[1]
            v = jnp.where(lane < 1, s0, s1)
            rowi = jnp.full((16,), i, jnp.int32)
            coli = jnp.minimum(lane, 1)
            plsc.store_scatter(ob[p], [rowi, coli], v, mask=lane < 2)
            return 0
        lax.fori_loop(0, CH, edge, 0, unroll=2)
        pltpu.async_copy(ob[p], out_hbm.at[pl.ds(g * EPT + j * CH, CH)],
                         semo[p])

    issue_g(0, 0)

    def body2(t, _):
        for half in range(2):
            p = half
            np = 1 - half
            j = 2 * t + half

            @pl.when(j < NCH)
            def _run():
                @pl.when(j + 1 < NCH)
                def _pf():
                    issue_g(j + 1, np)
                wait_g(p)

                @pl.when(j >= 2)
                def _wo():
                    wait_o(p)
                compute(j, p)
        return 0
    lax.fori_loop(0, (NCH + 1) // 2, body2, 0)
    wait_o(1)
    wait_o(0)


_final_edge = jax.jit(pl.kernel(
    _final_body,
    out_type=jax.ShapeDtypeStruct((E, 2), jnp.float32),
    mesh=_MESH,
    compiler_params=_SC_PARAMS,
    scratch_types=[
        pltpu.VMEM((NCH, CH), jnp.int32),
        pltpu.VMEM((NCH, CH), jnp.int32),
        pltpu.VMEM((CH, D), jnp.float32),
        pltpu.VMEM((CH, D), jnp.float32),
        pltpu.VMEM((CH, D), jnp.float32),
        pltpu.VMEM((CH, D), jnp.float32),
        pltpu.VMEM((CH, 2), jnp.float32),
        pltpu.VMEM((CH, 2), jnp.float32),
        pltpu.VMEM((2, D), jnp.float32),
        pltpu.VMEM((16,), jnp.float32),
        pltpu.SemaphoreType.DMA,
        pltpu.SemaphoreType.DMA,
        pltpu.SemaphoreType.DMA,
        pltpu.SemaphoreType.DMA,
    ],
))


def _pack_attn(al, ar, hds, od):
    al16 = jnp.zeros((D, 16), jnp.float32)
    ar16 = jnp.zeros((D, 16), jnp.float32)
    for h in range(hds):
        al16 = al16.at[h * od:(h + 1) * od, h].set(al[h])
        ar16 = ar16.at[h * od:(h + 1) * od, h].set(ar[h])
    sel = jnp.zeros((16, D), jnp.float32)
    for h in range(hds):
        sel = sel.at[h, h * od:(h + 1) * od].set(1.0)
    return al16, ar16, sel


def kernel(h, edge_index, W0, al0, ar0, g0, b0, W1, al1, ar1, g1, b1,
           W2, al2, ar2, g2, b2, W3, al3, ar3, g3, b3, P1, pb1, P2, pb2):
    src3 = edge_index[0].reshape(NTILES, NCH, CH)
    dst3 = edge_index[1].reshape(NTILES, NCH, CH)
    params = [(W0, al0, ar0, g0, b0), (W1, al1, ar1, g1, b1),
              (W2, al2, ar2, g2, b2), (W3, al3, ar3, g3, b3)]

    packed = [_pack_attn(al, ar, hds, od)
              for (ind, hds, od), (_, al, ar, _, _) in zip(LCFG, params)]

    hin = h
    al16, ar16, _ = packed[0]
    z, el16, er16 = _dense0(hin, W0, al16, ar16)
    for i in range(4):
        ind, hds, od = LCFG[i]
        _, _, _, g, b = params[i]
        edge_pass = _edge_pass_h8 if hds == 8 else _edge_pass_h1
        msum, esum16 = edge_pass(z, el16, er16, src3, dst3)
        sel = packed[i][2]
        if i < 3:
            al16n, ar16n, _ = packed[i + 1]
            hin, z, el16, er16 = _epi_dense(msum, esum16, sel, g, b, hin,
                                            params[i + 1][0], al16n, ar16n)
        else:
            a, bb = _epi_final(msum, esum16, sel, g, b, hin,
                               P1[:D], P1[D:], pb1)
    p2t = P2.T
    pb2p = jnp.zeros((16,), jnp.float32).at[:2].set(pb2)
    out = _final_edge(a, bb, src3, dst3, p2t, pb2p)
    return out


# speedup vs baseline: 111.1171x; 2.6091x over previous
import functools

import jax
import jax.numpy as jnp
from jax import lax
from jax.experimental import pallas as pl
from jax.experimental.pallas import tpu as pltpu
from jax.experimental.pallas import tpu_sc as plsc

N = 10000
E = 320000
D = 128
LCFG = [(128, 8, 16), (128, 8, 16), (128, 8, 16), (128, 1, 128)]

_MESH = plsc.VectorSubcoreMesh(core_axis_name="c", subcore_axis_name="s")
_SC_PARAMS = pltpu.CompilerParams(use_tc_tiling_on_sc=False,
                                  needs_layout_passes=False)
NTILES = 32
EPT = E // NTILES
CH = 80
NCH = EPT // CH
R0 = 624
REM = N - 16 * R0


def _vmem(n=None):
    return pl.BlockSpec(memory_space=pltpu.ANY) if n is None else pl.BlockSpec(memory_space=pltpu.VMEM)


def _dense0_body(h_ref, w_ref, al_ref, ar_ref, z_ref, el_ref, er_ref):
    z = jnp.dot(h_ref[...], w_ref[...], preferred_element_type=jnp.float32)
    z_ref[...] = z
    el_ref[...] = jnp.dot(z, al_ref[...], preferred_element_type=jnp.float32)
    er_ref[...] = jnp.dot(z, ar_ref[...], preferred_element_type=jnp.float32)


@jax.jit
def _dense0(h, w, al16, ar16):
    return pl.pallas_call(
        _dense0_body,
        out_shape=[
            jax.ShapeDtypeStruct((N, D), jnp.float32),
            jax.ShapeDtypeStruct((N, 16), jnp.float32),
            jax.ShapeDtypeStruct((N, 16), jnp.float32),
        ],
    )(h, w, al16, ar16)


def _epi_dense_body(ms_ref, es_ref, sel_ref, g_ref, b_ref, hin_ref,
                    w_ref, al_ref, ar_ref, h_ref, z_ref, el_ref, er_ref):
    es = es_ref[0] + es_ref[1]
    den = jnp.dot(es, sel_ref[...], preferred_element_type=jnp.float32)
    rst = (ms_ref[0] + ms_ref[1]) / (den + 1e-9)
    mean = jnp.mean(rst, axis=0, keepdims=True)
    var = jnp.mean(rst * rst, axis=0, keepdims=True) - mean * mean
    rst = (rst - mean) * jax.lax.rsqrt(var + 1e-5) * g_ref[...] + b_ref[...]
    rst = jnp.where(rst > 0, rst, jnp.exp(jnp.minimum(rst, 0.0)) - 1.0)
    hout = hin_ref[...] + rst
    h_ref[...] = hout
    z = jnp.dot(hout, w_ref[...], preferred_element_type=jnp.float32)
    z_ref[...] = z
    el_ref[...] = jnp.dot(z, al_ref[...], preferred_element_type=jnp.float32)
    er_ref[...] = jnp.dot(z, ar_ref[...], preferred_element_type=jnp.float32)


@jax.jit
def _epi_dense(msum, esum, sel, g, b, hin, w, al16, ar16):
    return pl.pallas_call(
        _epi_dense_body,
        out_shape=[
            jax.ShapeDtypeStruct((N, D), jnp.float32),
            jax.ShapeDtypeStruct((N, D), jnp.float32),
            jax.ShapeDtypeStruct((N, 16), jnp.float32),
            jax.ShapeDtypeStruct((N, 16), jnp.float32),
        ],
    )(msum, esum, sel, g.reshape(1, D), b.reshape(1, D), hin, w, al16, ar16)


def _epi_final_body(ms_ref, es_ref, sel_ref, g_ref, b_ref, hin_ref,
                    p1a_ref, p1b_ref, pb1_ref, a_ref, bout_ref):
    es = es_ref[0] + es_ref[1]
    den = jnp.dot(es, sel_ref[...], preferred_element_type=jnp.float32)
    rst = (ms_ref[0] + ms_ref[1]) / (den + 1e-9)
    mean = jnp.mean(rst, axis=0, keepdims=True)
    var = jnp.mean(rst * rst, axis=0, keepdims=True) - mean * mean
    rst = (rst - mean) * jax.lax.rsqrt(var + 1e-5) * g_ref[...] + b_ref[...]
    rst = jnp.where(rst > 0, rst, jnp.exp(jnp.minimum(rst, 0.0)) - 1.0)
    hout = hin_ref[...] + rst
    a_ref[...] = jnp.dot(hout, p1a_ref[...], preferred_element_type=jnp.float32)
    bout_ref[...] = (jnp.dot(hout, p1b_ref[...], preferred_element_type=jnp.float32)
                     + pb1_ref[...])


@jax.jit
def _epi_final(msum, esum, sel, g, b, hin, p1a, p1b, pb1):
    return pl.pallas_call(
        _epi_final_body,
        out_shape=[
            jax.ShapeDtypeStruct((N, D), jnp.float32),
            jax.ShapeDtypeStruct((N, D), jnp.float32),
        ],
    )(msum, esum, sel, g.reshape(1, D), b.reshape(1, D), hin, p1a, p1b,
      pb1.reshape(1, D))


def _zero16(ref, rows):
    def body(i, _):
        for t in range(ref.shape[1] // 16):
            ref[i, pl.ds(t * 16, 16)] = jnp.zeros((16,), jnp.float32)
        return 0
    lax.fori_loop(0, rows, body, 0, unroll=4)


def _edge_body(hds, od, z_hbm, el_hbm, er_hbm, src_hbm, dst_hbm,
               outm_hbm, oute_hbm,
               srcv0, srcv1, dstv0, dstv1, dsts, elg0, elg1, erg0, erg1,
               zg0, zg1, msg, exb,
               accm, acce, semi0, semi1, semg0, semg1):
    srcv = [srcv0, srcv1]
    dstv = [dstv0, dstv1]
    elg = [elg0, elg1]
    erg = [erg0, erg1]
    zg = [zg0, zg1]
    semi = [semi0, semi1]
    semg = [semg0, semg1]
    c = lax.axis_index("c")
    s = lax.axis_index("s")
    g = c * 16 + s
    _zero16(msg, CH)
    _zero16(exb, CH)
    base = s * R0
    nfull = R0 // CH
    rem = R0 - nfull * CH
    for k in range(nfull):
        pltpu.sync_copy(msg, accm.at[pl.ds(base + k * CH, CH)])
        pltpu.sync_copy(exb, acce.at[pl.ds(base + k * CH, CH)])
    if rem:
        pltpu.sync_copy(msg.at[pl.ds(0, rem)],
                        accm.at[pl.ds(base + nfull * CH, rem)])
        pltpu.sync_copy(exb.at[pl.ds(0, rem)],
                        acce.at[pl.ds(base + nfull * CH, rem)])

    @pl.when(s == 15)
    def _zero_tail():
        pltpu.sync_copy(msg.at[pl.ds(0, REM)], accm.at[pl.ds(16 * R0, REM)])
        pltpu.sync_copy(exb.at[pl.ds(0, REM)], acce.at[pl.ds(16 * R0, REM)])
    plsc.subcore_barrier()

    def issue_idx(j, p):
        pltpu.async_copy(src_hbm.at[g].at[j], srcv[p], semi[p])
        pltpu.async_copy(dst_hbm.at[g].at[j], dstv[p], semi[p])

    def wait_idx(p):
        pltpu.make_async_copy(src_hbm.at[g].at[0], srcv[p], semi[p]).wait()
        pltpu.make_async_copy(dst_hbm.at[g].at[0], dstv[p], semi[p]).wait()

    def issue_gathers(p):
        pltpu.async_copy(el_hbm.at[srcv[p]], elg[p], semg[p])
        pltpu.async_copy(er_hbm.at[dstv[p]], erg[p], semg[p])
        pltpu.async_copy(z_hbm.at[srcv[p]], zg[p], semg[p])

    def wait_gathers(p):
        pltpu.make_async_copy(el_hbm.at[srcv[p]], elg[p], semg[p]).wait()
        pltpu.make_async_copy(er_hbm.at[dstv[p]], erg[p], semg[p]).wait()
        pltpu.make_async_copy(z_hbm.at[srcv[p]], zg[p], semg[p]).wait()

    gdn = lax.GatherDimensionNumbers(offset_dims=(), collapsed_slice_dims=(0,),
                                     start_index_map=(0,))
    idxh = [jnp.full((16, 1), h, jnp.int32) for h in range(hds)]

    def compute_scatter(p):
        @plsc.parallel_loop(0, CH, unroll=4)
        def edge(i):
            a = elg[p][i, :] + erg[p][i, :]
            a = jnp.maximum(a, 0.2 * a)
            ex = jnp.exp(a)
            exb[i, :] = ex
            for h in range(hds):
                bv = lax.gather(ex, idxh[h], gdn, slice_sizes=(1,),
                                mode=lax.GatherScatterMode.PROMISE_IN_BOUNDS)
                for t in range(od // 16):
                    off = h * od + t * 16
                    msg[i, pl.ds(off, 16)] = bv * zg[p][i, pl.ds(off, 16)]
        pltpu.sync_copy(msg, accm.at[dsts], add=True)
        pltpu.sync_copy(exb, acce.at[dsts], add=True)

    pltpu.sync_copy(src_hbm.at[g].at[0], srcv[0])
    pltpu.sync_copy(dst_hbm.at[g].at[0], dstv[0])
    issue_gathers(0)
    issue_idx(1, 1)

    def body2(t, _):
        for half in range(2):
            p = half
            np = 1 - half
            j = 2 * t + half

            @pl.when(j < NCH)
            def _run():
                @pl.when(j + 1 < NCH)
                def _pf():
                    wait_idx(np)
                    issue_gathers(np)
                wait_gathers(p)
                for k in range(CH // 16):
                    dsts[pl.ds(k * 16, 16)] = dstv[p][pl.ds(k * 16, 16)]

                @pl.when(j + 2 < NCH)
                def _pi():
                    issue_idx(j + 2, p)
                compute_scatter(p)
        return 0
    lax.fori_loop(0, (NCH + 1) // 2, body2, 0)
    plsc.subcore_barrier()
    pltpu.sync_copy(accm.at[pl.ds(base, R0)],
                    outm_hbm.at[c].at[pl.ds(base, R0)])
    pltpu.sync_copy(acce.at[pl.ds(base, R0)],
                    oute_hbm.at[c].at[pl.ds(base, R0)])

    @pl.when(s == 15)
    def _out_tail():
        pltpu.sync_copy(accm.at[pl.ds(16 * R0, REM)],
                        outm_hbm.at[c].at[pl.ds(16 * R0, REM)])
        pltpu.sync_copy(acce.at[pl.ds(16 * R0, REM)],
                        oute_hbm.at[c].at[pl.ds(16 * R0, REM)])


def _make_edge_pass(hds, od):
    return pl.kernel(
        functools.partial(_edge_body, hds, od),
        out_type=[
            jax.ShapeDtypeStruct((2, N, D), jnp.float32),
            jax.ShapeDtypeStruct((2, N, 16), jnp.float32),
        ],
        mesh=_MESH,
        compiler_params=_SC_PARAMS,
        scratch_types=[
            pltpu.VMEM((CH,), jnp.int32),
            pltpu.VMEM((CH,), jnp.int32),
            pltpu.VMEM((CH,), jnp.int32),
            pltpu.VMEM((CH,), jnp.int32),
            pltpu.VMEM((CH,), jnp.int32),
            pltpu.VMEM((CH, 16), jnp.float32),
            pltpu.VMEM((CH, 16), jnp.float32),
            pltpu.VMEM((CH, 16), jnp.float32),
            pltpu.VMEM((CH, 16), jnp.float32),
            pltpu.VMEM((CH, D), jnp.float32),
            pltpu.VMEM((CH, D), jnp.float32),
            pltpu.VMEM((CH, D), jnp.float32),
            pltpu.VMEM((CH, 16), jnp.float32),
            pltpu.VMEM_SHARED((N, D), jnp.float32),
            pltpu.VMEM_SHARED((N, 16), jnp.float32),
            pltpu.SemaphoreType.DMA,
            pltpu.SemaphoreType.DMA,
            pltpu.SemaphoreType.DMA,
            pltpu.SemaphoreType.DMA,
        ],
    )


_edge_pass_h8 = jax.jit(_make_edge_pass(8, 16))
_edge_pass_h1 = jax.jit(_make_edge_pass(1, 128))


def _final_body(a_hbm, b_hbm, src_hbm, dst_hbm, p2_hbm, pb2_hbm, out_hbm,
                srcv, dstv, ag0, ag1, bg0, bg1, ob0, ob1, p2v, pb2v,
                semg0, semg1, semo0, semo1):
    ag = [ag0, ag1]
    bg = [bg0, bg1]
    ob = [ob0, ob1]
    semg = [semg0, semg1]
    semo = [semo0, semo1]
    c = lax.axis_index("c")
    s = lax.axis_index("s")
    g = c * 16 + s
    pltpu.sync_copy(src_hbm.at[g], srcv)
    pltpu.sync_copy(dst_hbm.at[g], dstv)
    pltpu.sync_copy(p2_hbm, p2v)
    pltpu.sync_copy(pb2_hbm, pb2v)
    pb2vec = pb2v[:]
    lane = lax.iota(jnp.int32, 16)

    def issue_g(j, p):
        pltpu.async_copy(a_hbm.at[srcv.at[j]], ag[p], semg[p])
        pltpu.async_copy(b_hbm.at[dstv.at[j]], bg[p], semg[p])

    def wait_g(p):
        pltpu.make_async_copy(a_hbm.at[srcv.at[0]], ag[p], semg[p]).wait()
        pltpu.make_async_copy(b_hbm.at[dstv.at[0]], bg[p], semg[p]).wait()

    def wait_o(p):
        pltpu.make_async_copy(ob[p], out_hbm.at[pl.ds(g * EPT, CH)],
                              semo[p]).wait()

    def compute(j, p):
        @plsc.parallel_loop(0, CH, unroll=4)
        def edge(i):
            acc0 = jnp.zeros((16,), jnp.float32)
            acc1 = jnp.zeros((16,), jnp.float32)
            for t in range(D // 16):
                r = jnp.maximum(ag[p][i, pl.ds(t * 16, 16)]
                                + bg[p][i, pl.ds(t * 16, 16)], 0.0)
                acc0 = acc0 + r * p2v[0, pl.ds(t * 16, 16)]
                acc1 = acc1 + r * p2v[1, pl.ds(t * 16, 16)]
            s0 = jnp.sum(acc0) + pb2vec[0]
            s1 = jnp.sum(acc1) + pb2vec[1]
            v = jnp.where(lane < 1, s0, s1)
            rowi = jnp.full((16,), i, jnp.int32)
            coli = jnp.minimum(lane, 1)
            plsc.store_scatter(ob[p], [rowi, coli], v, mask=lane < 2)
        pltpu.async_copy(ob[p], out_hbm.at[pl.ds(g * EPT + j * CH, CH)],
                         semo[p])

    issue_g(0, 0)

    def body2(t, _):
        for half in range(2):
            p = half
            np = 1 - half
            j = 2 * t + half

            @pl.when(j < NCH)
            def _run():
                @pl.when(j + 1 < NCH)
                def _pf():
                    issue_g(j + 1, np)
                wait_g(p)

                @pl.when(j >= 2)
                def _wo():
                    wait_o(p)
                compute(j, p)
        return 0
    lax.fori_loop(0, (NCH + 1) // 2, body2, 0)
    wait_o(1)
    wait_o(0)


_final_edge = jax.jit(pl.kernel(
    _final_body,
    out_type=jax.ShapeDtypeStruct((E, 2), jnp.float32),
    mesh=_MESH,
    compiler_params=_SC_PARAMS,
    scratch_types=[
        pltpu.VMEM((NCH, CH), jnp.int32),
        pltpu.VMEM((NCH, CH), jnp.int32),
        pltpu.VMEM((CH, D), jnp.float32),
        pltpu.VMEM((CH, D), jnp.float32),
        pltpu.VMEM((CH, D), jnp.float32),
        pltpu.VMEM((CH, D), jnp.float32),
        pltpu.VMEM((CH, 2), jnp.float32),
        pltpu.VMEM((CH, 2), jnp.float32),
        pltpu.VMEM((2, D), jnp.float32),
        pltpu.VMEM((16,), jnp.float32),
        pltpu.SemaphoreType.DMA,
        pltpu.SemaphoreType.DMA,
        pltpu.SemaphoreType.DMA,
        pltpu.SemaphoreType.DMA,
    ],
))


def _pack_attn(al, ar, hds, od):
    al16 = jnp.zeros((D, 16), jnp.float32)
    ar16 = jnp.zeros((D, 16), jnp.float32)
    for h in range(hds):
        al16 = al16.at[h * od:(h + 1) * od, h].set(al[h])
        ar16 = ar16.at[h * od:(h + 1) * od, h].set(ar[h])
    sel = jnp.zeros((16, D), jnp.float32)
    for h in range(hds):
        sel = sel.at[h, h * od:(h + 1) * od].set(1.0)
    return al16, ar16, sel


def kernel(h, edge_index, W0, al0, ar0, g0, b0, W1, al1, ar1, g1, b1,
           W2, al2, ar2, g2, b2, W3, al3, ar3, g3, b3, P1, pb1, P2, pb2):
    src3 = edge_index[0].reshape(NTILES, NCH, CH)
    dst3 = edge_index[1].reshape(NTILES, NCH, CH)
    params = [(W0, al0, ar0, g0, b0), (W1, al1, ar1, g1, b1),
              (W2, al2, ar2, g2, b2), (W3, al3, ar3, g3, b3)]

    packed = [_pack_attn(al, ar, hds, od)
              for (ind, hds, od), (_, al, ar, _, _) in zip(LCFG, params)]

    hin = h
    al16, ar16, _ = packed[0]
    z, el16, er16 = _dense0(hin, W0, al16, ar16)
    for i in range(4):
        ind, hds, od = LCFG[i]
        _, _, _, g, b = params[i]
        edge_pass = _edge_pass_h8 if hds == 8 else _edge_pass_h1
        msum, esum16 = edge_pass(z, el16, er16, src3, dst3)
        sel = packed[i][2]
        if i < 3:
            al16n, ar16n, _ = packed[i + 1]
            hin, z, el16, er16 = _epi_dense(msum, esum16, sel, g, b, hin,
                                            params[i + 1][0], al16n, ar16n)
        else:
            a, bb = _epi_final(msum, esum16, sel, g, b, hin,
                               P1[:D], P1[D:], pb1)
    p2t = P2.T
    pb2p = jnp.zeros((16,), jnp.float32).at[:2].set(pb2)
    out = _final_edge(a, bb, src3, dst3, p2t, pb2p)
    return out
